# baseline (speedup 1.0000x reference)
import functools

import jax
import jax.numpy as jnp
from jax import lax
from jax.experimental import pallas as pl
from jax.experimental.pallas import tpu as pltpu
from jax.experimental.pallas import tpu_sc as plsc

try:
    _info = plsc.get_sparse_core_info()
    NC, NS, NL = _info.num_cores, _info.num_subcores, _info.num_lanes
except Exception:
    NC, NS, NL = 2, 16, 16
NW = NC * NS


def _make_sc_encode(B, ncat, nnum, D, V):
    rows_cat = ncat * D
    rows_num = nnum * D
    rows_all = rows_num + rows_cat
    ntasks = (rows_all + NW - 1) // NW
    nb = 4096
    nchunks = B // nb
    mesh = plsc.VectorSubcoreMesh(core_axis_name="c", subcore_axis_name="s")

    n_cat_w = rows_cat // NW
    n_num_hi = -(-rows_num // NW)
    n_chunk = nb

    @functools.partial(
        pl.kernel,
        out_type=jax.ShapeDtypeStruct((rows_all, B), jnp.float32),
        mesh=mesh,
        scratch_types=[
            pltpu.VMEM((V,), jnp.float32),
            pltpu.VMEM((B,), jnp.int32),
            pltpu.VMEM((nb,), jnp.float32),
            pltpu.VMEM((nb,), jnp.float32),
            pltpu.VMEM((rows_num + NL,), jnp.float32),
            pltpu.VMEM((rows_num + NL,), jnp.float32),
            pltpu.SemaphoreType.DMA,
            pltpu.SemaphoreType.DMA,
            pltpu.SemaphoreType.DMA,
            pltpu.SemaphoreType.DMA,
        ],
        compiler_params=pltpu.CompilerParams(
            use_tc_tiling_on_sc=True, needs_layout_passes=False),
    )
    def sc_encode(table_hbm, idx_hbm, vals_hbm, wflat_hbm, bflat_hbm, out_hbm,
                  trow_v, idx_v, out0_v, out1_v, w_v, b_v,
                  sem_row, sem_idx, sem_o0, sem_o1):
        wid = lax.axis_index("s") * NC + lax.axis_index("c")
        pltpu.sync_copy(wflat_hbm, w_v.at[pl.ds(0, rows_num)])
        pltpu.sync_copy(bflat_hbm, b_v.at[pl.ds(0, rows_num)])
        out_bufs = (out0_v, out1_v)
        out_sems = (sem_o0, sem_o1)
        n0 = n_num_hi * wid - jnp.maximum(
            wid - (NW - (n_num_hi * NW - rows_num)), 0)
        ncount = jnp.where(
            wid >= NW - (n_num_hi * NW - rows_num), n_num_hi - 1, n_num_hi)
        r0 = n_cat_w * wid

        def drain(par):
            pltpu.make_async_copy(
                out_bufs[par], out_hbm.at[0].at[pl.ds(0, nb)],
                out_sems[par]).wait()

        def cat_task(i, h_row, first):
            r = r0 + i
            c = rows_num + r
            if h_row is None:
                h_row = pltpu.async_copy(table_hbm.at[r], trow_v, sem_row)
            if i == 0:
                pltpu.async_copy(
                    idx_hbm.at[r // D], idx_v, sem_idx).wait()
            else:
                @pl.when((r // D) != ((r - 1) // D))
                def _():
                    pltpu.async_copy(
                        idx_hbm.at[r // D], idx_v, sem_idx).wait()

            h_row.wait()
            for k in range(nchunks):
                if not (first and k < 2):
                    drain(k % 2)
                ob = out_bufs[k % 2]
                base = k * n_chunk

                def vec_body(ii, carry, _base=base, _ob=ob):
                    off = ii * (8 * NL)
                    for u in range(8):
                        s = pl.ds(
                            pl.multiple_of(_base + off + u * NL, NL), NL)
                        so = pl.ds(pl.multiple_of(off + u * NL, NL), NL)
                        _ob[so] = plsc.load_gather(trow_v, [idx_v[s]])
                    return carry

                lax.fori_loop(0, nb // (8 * NL), vec_body, 0)
                pltpu.async_copy(
                    ob, out_hbm.at[c].at[pl.ds(k * n_chunk, n_chunk)],
                    out_sems[k % 2])

        def num_task(i, first):
            c = n0 + i
            j = c // D
            w = w_v[pl.ds(c, NL)][0]
            bb = b_v[pl.ds(c, NL)][0]
            for k in range(nchunks):
                if not (first and k < 2):
                    drain(k % 2)
                ob = out_bufs[k % 2]
                pltpu.async_copy(
                    vals_hbm.at[j].at[pl.ds(k * n_chunk, n_chunk)],
                    ob, sem_idx).wait()

                def vec_body(ii, carry, _ob=ob):
                    off = ii * (4 * NL)
                    for u in range(4):
                        s = pl.ds(pl.multiple_of(off + u * NL, NL), NL)
                        x = _ob[s]
                        nanm = x != x
                        xm = jnp.where(nanm, jnp.float32(0.0), x)
                        y = jnp.maximum(xm * w + bb, jnp.float32(0.0))
                        _ob[s] = jnp.where(nanm, jnp.float32(0.0), y)
                    return carry

                lax.fori_loop(0, nb // (4 * NL), vec_body, 0)
                pltpu.async_copy(
                    ob, out_hbm.at[c].at[pl.ds(k * n_chunk, n_chunk)],
                    out_sems[k % 2])

        pending = None
        for i in range(n_num_hi):
            pending = pltpu.async_copy(
                table_hbm.at[r0 + i], trow_v, sem_row)
            if i == 0:
                num_task(i, first=True)
            else:
                @pl.when(i < ncount)
                def _(i=i):
                    num_task(i, first=False)
            cat_task(i, pending, first=False)
        for i in range(n_num_hi, n_cat_w):
            cat_task(i, None, first=False)
        drain(0)
        drain(1)

    return sc_encode


def kernel(num_values, cat_indices, num_W, num_b, cat_tables):
    B, nnum = num_values.shape
    _, ncat = cat_indices.shape
    V, D = cat_tables.shape[1], cat_tables.shape[2]

    table2 = cat_tables.transpose(0, 2, 1).reshape(ncat * D, V)
    idx_t = cat_indices.T
    vals_t = num_values.T
    wflat = num_W.reshape(nnum * D)
    bflat = num_b.reshape(nnum * D)

    out_t = _make_sc_encode(B, ncat, nnum, D, V)(
        table2, idx_t, vals_t, wflat, bflat)
    return out_t.T


# speedup vs baseline: 1.3146x; 1.3146x over previous
import functools

import jax
import jax.numpy as jnp
from jax import lax
from jax.experimental import pallas as pl
from jax.experimental.pallas import tpu as pltpu
from jax.experimental.pallas import tpu_sc as plsc

try:
    _info = plsc.get_sparse_core_info()
    NC, NS, NL = _info.num_cores, _info.num_subcores, _info.num_lanes
except Exception:
    NC, NS, NL = 2, 16, 16
NW = NC * NS


class _SplitRowCopy:
    def __init__(self, handles):
        self.handles = handles

    def wait(self):
        for h in self.handles:
            h.wait()


_ROW_SPLIT = 1


def _row_copy(table_hbm, r, trow_v, sem):
    return _SplitRowCopy([pltpu.async_copy(table_hbm.at[r], trow_v, sem)])


def _make_sc_encode(B, ncat, nnum, D, V):
    rows_cat = ncat * D
    rows_num = nnum * D
    rows_all = rows_num + rows_cat
    ntasks = (rows_all + NW - 1) // NW
    nb = 4096
    nchunks = B // nb
    mesh = plsc.VectorSubcoreMesh(core_axis_name="c", subcore_axis_name="s")

    n_cat_w = rows_cat // NW
    n_num_hi = -(-rows_num // NW)
    n_chunk = nb

    @functools.partial(
        pl.kernel,
        out_type=jax.ShapeDtypeStruct((rows_all, B), jnp.float32),
        mesh=mesh,
        scratch_types=[
            pltpu.VMEM((V,), jnp.float32),
            pltpu.VMEM((B,), jnp.int32),
            pltpu.VMEM((nb,), jnp.float32),
            pltpu.VMEM((nb,), jnp.float32),
            pltpu.VMEM((rows_num + NL,), jnp.float32),
            pltpu.VMEM((rows_num + NL,), jnp.float32),
            pltpu.SemaphoreType.DMA,
            pltpu.SemaphoreType.DMA,
            pltpu.SemaphoreType.DMA,
            pltpu.SemaphoreType.DMA,
        ],
        compiler_params=pltpu.CompilerParams(
            use_tc_tiling_on_sc=True, needs_layout_passes=False),
    )
    def sc_encode(table_hbm, idx_hbm, vals_hbm, wflat_hbm, bflat_hbm, out_hbm,
                  trow_v, idx_v, out0_v, out1_v, w_v, b_v,
                  sem_row, sem_idx, sem_o0, sem_o1):
        wid = lax.axis_index("s") * NC + lax.axis_index("c")
        pltpu.sync_copy(wflat_hbm, w_v.at[pl.ds(0, rows_num)])
        pltpu.sync_copy(bflat_hbm, b_v.at[pl.ds(0, rows_num)])
        out_bufs = (out0_v, out1_v)
        out_sems = (sem_o0, sem_o1)
        n0 = n_num_hi * wid - jnp.maximum(
            wid - (NW - (n_num_hi * NW - rows_num)), 0)
        ncount = jnp.where(
            wid >= NW - (n_num_hi * NW - rows_num), n_num_hi - 1, n_num_hi)
        r0 = n_cat_w * wid

        def drain(par):
            pltpu.make_async_copy(
                out_bufs[par], out_hbm.at[0].at[pl.ds(0, nb)],
                out_sems[par]).wait()

        def cat_task(i, h_row, first):
            r = r0 + i
            c = rows_num + r
            if h_row is None:
                h_row = _row_copy(table_hbm, r, trow_v, sem_row)
            if i == 0:
                pltpu.async_copy(
                    idx_hbm.at[r // D], idx_v, sem_idx).wait()
            else:
                @pl.when((r // D) != ((r - 1) // D))
                def _():
                    pltpu.async_copy(
                        idx_hbm.at[r // D], idx_v, sem_idx).wait()

            h_row.wait()
            for k in range(nchunks):
                if not (first and k < 2):
                    drain(k % 2)
                ob = out_bufs[k % 2]
                base = k * n_chunk

                @plsc.parallel_loop(0, nb, step=NL, unroll=8)
                def vec_body(ii, _base=base, _ob=ob):
                    s = pl.ds(pl.multiple_of(_base + ii, NL), NL)
                    so = pl.ds(pl.multiple_of(ii, NL), NL)
                    _ob[so] = plsc.load_gather(trow_v, [idx_v[s]])
                pltpu.async_copy(
                    ob, out_hbm.at[c].at[pl.ds(k * n_chunk, n_chunk)],
                    out_sems[k % 2])

        def num_task(i, first):
            c = n0 + i
            j = c // D
            w = w_v[pl.ds(c, NL)][0]
            bb = b_v[pl.ds(c, NL)][0]
            for k in range(nchunks):
                if not (first and k < 2):
                    drain(k % 2)
                ob = out_bufs[k % 2]
                pltpu.async_copy(
                    vals_hbm.at[j].at[pl.ds(k * n_chunk, n_chunk)],
                    ob, sem_idx).wait()

                @plsc.parallel_loop(0, nb, step=NL, unroll=4)
                def vec_body(ii, _ob=ob):
                    s = pl.ds(pl.multiple_of(ii, NL), NL)
                    x = _ob[s]
                    nanm = x != x
                    xm = jnp.where(nanm, jnp.float32(0.0), x)
                    y = jnp.maximum(xm * w + bb, jnp.float32(0.0))
                    _ob[s] = jnp.where(nanm, jnp.float32(0.0), y)
                pltpu.async_copy(
                    ob, out_hbm.at[c].at[pl.ds(k * n_chunk, n_chunk)],
                    out_sems[k % 2])

        pending = None
        for i in range(n_num_hi):
            pending = _row_copy(table_hbm, r0 + i, trow_v, sem_row)
            if i == 0:
                num_task(i, first=True)
            else:
                @pl.when(i < ncount)
                def _(i=i):
                    num_task(i, first=False)
            cat_task(i, pending, first=False)
        for i in range(n_num_hi, n_cat_w):
            cat_task(i, None, first=False)
        drain(0)
        drain(1)

    return sc_encode


def kernel(num_values, cat_indices, num_W, num_b, cat_tables):
    B, nnum = num_values.shape
    _, ncat = cat_indices.shape
    V, D = cat_tables.shape[1], cat_tables.shape[2]

    table2 = cat_tables.transpose(0, 2, 1).reshape(ncat * D, V)
    idx_t = cat_indices.T
    vals_t = num_values.T
    wflat = num_W.reshape(nnum * D)
    bflat = num_b.reshape(nnum * D)

    out_t = _make_sc_encode(B, ncat, nnum, D, V)(
        table2, idx_t, vals_t, wflat, bflat)
    return out_t.T
